# Initial kernel scaffold; baseline (speedup 1.0000x reference)
import jax
import jax.numpy as jnp
from jax.experimental import pallas as pl


def kernel(input, target, one_hot):
    raise NotImplementedError("write your pallas kernel here")


# speedup vs baseline: 1.0986x; 1.0986x over previous
import functools

import jax
import jax.numpy as jnp
from jax import lax
from jax.experimental import pallas as pl
from jax.experimental.pallas import tpu as pltpu
from jax.experimental.pallas import tpu_sc as plsc

_PAD = 0
_BOS = 2
_N = 2048
_C = 100000
_CB = 2048
_NBJ = pl.cdiv(_C, _CB)
_NW = 32
_BPW = _N // _NW
_L = 16


def _dense_body(x_ref, t_ref, out_ref):
    j = pl.program_id(0)
    nonpad = t_ref[...] != _PAD

    @pl.when(j == 0)
    def _():
        x = x_ref[...]
        rs = jnp.sum(x, axis=1, keepdims=True) - x[:, 0:1] - x[:, 2:3]
        out_ref[0] = jnp.sum(jnp.where(nonpad, rs, 0.0))

    @pl.when(jnp.logical_and(j > 0, j < _NBJ - 1))
    def _():
        rs = jnp.sum(x_ref[...], axis=1, keepdims=True)
        out_ref[0] += jnp.sum(jnp.where(nonpad, rs, 0.0))

    @pl.when(j == _NBJ - 1)
    def _():
        cols = lax.broadcasted_iota(jnp.int32, (_N, _CB), 1) + j * _CB
        xm = jnp.where(cols < _C, x_ref[...], 0.0)
        rs = jnp.sum(xm, axis=1, keepdims=True)
        out_ref[0] += jnp.sum(jnp.where(nonpad, rs, 0.0))


_dense_sum = pl.pallas_call(
    _dense_body,
    grid=(_NBJ,),
    in_specs=[
        pl.BlockSpec((_N, _CB), lambda j: (0, j)),
        pl.BlockSpec((_N, 1), lambda j: (0, 0)),
    ],
    out_specs=pl.BlockSpec(memory_space=pltpu.SMEM),
    out_shape=jax.ShapeDtypeStruct((1,), jnp.float32),
)


def _gather_body(inp_ref, tgt_ref, out_ref, tgt_v, idx_v, vals_v, acc_v, sem):
    cid = lax.axis_index("c")
    sid = lax.axis_index("s")
    wid = sid * 2 + cid
    base = wid * _BPW
    pltpu.sync_copy(tgt_ref.at[pl.ds(base, _BPW)], tgt_v)
    iota = lax.iota(jnp.int32, _L)
    for c in range(_BPW // _L):
        t = tgt_v[pl.ds(c * _L, _L)]
        r = base + c * _L + iota
        idx_v[pl.ds(c * _L, _L)] = r * _C + t
    pltpu.async_copy(inp_ref.at[idx_v], vals_v, sem).wait()
    g = jnp.zeros((_L,), jnp.float32)
    g2 = jnp.zeros((_L,), jnp.float32)
    cnp = jnp.zeros((_L,), jnp.float32)
    cn2 = jnp.zeros((_L,), jnp.float32)
    one = jnp.ones((_L,), jnp.float32)
    zero = jnp.zeros((_L,), jnp.float32)
    for c in range(_BPW // _L):
        t = tgt_v[pl.ds(c * _L, _L)]
        vals = vals_v[pl.ds(c * _L, _L)]
        nonpad = t != _PAD
        vals = jnp.where(nonpad, vals, zero)
        is2 = t == _BOS
        g = g + vals
        g2 = g2 + jnp.where(is2, vals, zero)
        cnp = cnp + jnp.where(nonpad, one, zero)
        cn2 = cn2 + jnp.where(is2, one, zero)
    acc_v[pl.ds(0 * _L, _L)] = g
    acc_v[pl.ds(1 * _L, _L)] = g2
    acc_v[pl.ds(2 * _L, _L)] = cnp
    acc_v[pl.ds(3 * _L, _L)] = cn2
    pltpu.sync_copy(acc_v, out_ref.at[wid])


@functools.cache
def _gather_partials():
    return functools.partial(
        pl.kernel,
        mesh=plsc.VectorSubcoreMesh(core_axis_name="c", subcore_axis_name="s"),
        out_type=jax.ShapeDtypeStruct((_NW, 4 * _L), jnp.float32),
        scratch_types=[
            pltpu.VMEM((_BPW,), jnp.int32),
            pltpu.VMEM((_BPW,), jnp.int32),
            pltpu.VMEM((_BPW,), jnp.float32),
            pltpu.VMEM((4 * _L,), jnp.float32),
            pltpu.SemaphoreType.DMA,
        ],
    )(_gather_body)


@jax.jit
def kernel(input, target, one_hot):
    t2d = target.reshape(_N, 1).astype(jnp.int32)
    s_ex = _dense_sum(input, t2d)[0]
    inp_flat = input.reshape(_N * _C)
    partials = _gather_partials()(inp_flat, target.astype(jnp.int32))
    g = jnp.sum(partials[:, 0 * _L:1 * _L])
    g2 = jnp.sum(partials[:, 1 * _L:2 * _L])
    n_np = jnp.sum(partials[:, 2 * _L:3 * _L])
    n_2 = jnp.sum(partials[:, 3 * _L:4 * _L])
    eps = one_hot[1]
    conf = 1.0 - eps * (_C - 2)
    loss = (n_np * conf * jnp.log(conf)
            + eps * jnp.log(eps) * ((_C - 3) * n_np + n_2)
            - (eps * s_ex - eps * (g - g2) + conf * g))
    nll = -g
    return loss, nll


# speedup vs baseline: 2.4234x; 2.2060x over previous
import functools

import jax
import jax.numpy as jnp
from jax import lax
from jax.experimental import pallas as pl
from jax.experimental.pallas import tpu as pltpu

_PAD = 0
_BOS = 2
_N = 2048
_C = 100000
_CB = 2048
_NBJ = pl.cdiv(_C, _CB)


def _dense_body(x_ref, t_ref, out_ref):
    j = pl.program_id(0)
    t = t_ref[...]
    nonpad = t != _PAD
    cols = lax.broadcasted_iota(jnp.int32, (_N, _CB), 1) + j * _CB
    x = x_ref[...]
    match = cols == t
    gv = jnp.sum(jnp.where(match, x, 0.0), axis=1, keepdims=True)
    gvm = jnp.where(nonpad, gv, 0.0)

    @pl.when(j == 0)
    def _():
        rs = jnp.sum(x, axis=1, keepdims=True) - x[:, 0:1] - x[:, 2:3]
        out_ref[0] = jnp.sum(jnp.where(nonpad, rs, 0.0))
        out_ref[1] = jnp.sum(gvm)
        out_ref[2] = jnp.sum(jnp.where(t == _BOS, gvm, 0.0))
        out_ref[3] = jnp.sum(jnp.where(nonpad, 1.0, 0.0))
        out_ref[4] = jnp.sum(jnp.where(t == _BOS, 1.0, 0.0))

    @pl.when(jnp.logical_and(j > 0, j < _NBJ - 1))
    def _():
        rs = jnp.sum(x, axis=1, keepdims=True)
        out_ref[0] += jnp.sum(jnp.where(nonpad, rs, 0.0))
        out_ref[1] += jnp.sum(gvm)

    @pl.when(j == _NBJ - 1)
    def _():
        xm = jnp.where(cols < _C, x, 0.0)
        rs = jnp.sum(xm, axis=1, keepdims=True)
        out_ref[0] += jnp.sum(jnp.where(nonpad, rs, 0.0))
        out_ref[1] += jnp.sum(gvm)


_dense_sums = pl.pallas_call(
    _dense_body,
    grid=(_NBJ,),
    in_specs=[
        pl.BlockSpec((_N, _CB), lambda j: (0, j)),
        pl.BlockSpec((_N, 1), lambda j: (0, 0)),
    ],
    out_specs=pl.BlockSpec(memory_space=pltpu.SMEM),
    out_shape=jax.ShapeDtypeStruct((5,), jnp.float32),
)


@jax.jit
def kernel(input, target, one_hot):
    t2d = target.reshape(_N, 1).astype(jnp.int32)
    sums = _dense_sums(input, t2d)
    s_ex, g, g2, n_np, n_2 = sums[0], sums[1], sums[2], sums[3], sums[4]
    eps = one_hot[1]
    conf = 1.0 - eps * (_C - 2)
    loss = (n_np * conf * jnp.log(conf)
            + eps * jnp.log(eps) * ((_C - 3) * n_np + n_2)
            - (eps * s_ex - eps * (g - g2) + conf * g))
    nll = -g
    return loss, nll
